# baseline (speedup 1.0000x reference)
import functools

import jax
import jax.numpy as jnp
from jax import lax
from jax.experimental import pallas as pl
from jax.experimental.pallas import tpu as pltpu
from jax.experimental.pallas import tpu_sc as plsc

N = 10000
E = 320000
C = 128
G = 64
OUT = 64

NC = 2
NS = 16
NW = NC * NS
K = 128
NCHUNK = 80
EPW = NCHUNK * K
EPAD = NW * EPW
HCH = NCHUNK // 2
NPAD = 10240
RPT = NPAD // NS

_mesh = plsc.VectorSubcoreMesh(core_axis_name="c", subcore_axis_name="s")


@functools.partial(
    pl.kernel,
    out_type=jax.ShapeDtypeStruct((NC, NPAD, C), jnp.float32),
    mesh=_mesh,
    scratch_types=[
        pltpu.VMEM((K, C), jnp.float32),
        pltpu.VMEM((NCHUNK, K), jnp.int32),
        pltpu.VMEM_SHARED((NPAD, C), jnp.float32),
        pltpu.SemaphoreType.DMA,
    ],
)
def _deg(dst3_hbm, out_hbm, ones_v, didx, deg_sh, sem):
    c = lax.axis_index("c")
    s = lax.axis_index("s")
    wid = c * NS + s
    z16 = jnp.zeros((16,), jnp.float32)
    ones16 = jnp.full((16,), 1.0, jnp.float32)

    @pl.loop(0, K)
    def _zero(i):
        for j in range(C // 16):
            ones_v[i, pl.ds(j * 16, 16)] = z16

    @pl.loop(0, RPT // K)
    def _zcopy(j):
        pltpu.sync_copy(ones_v, deg_sh.at[pl.ds(s * RPT + j * K, K)])

    pltpu.sync_copy(dst3_hbm.at[wid], didx)
    plsc.subcore_barrier()

    @pl.loop(0, K)
    def _ones(i):
        for j in range(C // 16):
            ones_v[i, pl.ds(j * 16, 16)] = ones16

    GD = 5
    @pl.loop(0, NCHUNK // GD)
    def _grp(g):
        for t in range(GD):
            pltpu.async_copy(ones_v, deg_sh.at[didx.at[g * GD + t]], sem,
                             add=True)
        for t in range(GD):
            pltpu.make_async_copy(ones_v, deg_sh.at[didx.at[g * GD + t]],
                                  sem).wait()

    plsc.subcore_barrier()
    pltpu.sync_copy(deg_sh.at[pl.ds(s * RPT, RPT)],
                    out_hbm.at[c, pl.ds(s * RPT, RPT)])


@functools.partial(
    pl.kernel,
    out_type=jax.ShapeDtypeStruct((NC, NPAD, C), jnp.float32),
    mesh=_mesh,
    scratch_types=[
        pltpu.VMEM((K, C), jnp.float32),
        pltpu.VMEM((K, C), jnp.float32),
        pltpu.VMEM((HCH * K,), jnp.int32),
        pltpu.VMEM((NCHUNK, K), jnp.int32),
        pltpu.VMEM_SHARED((NPAD, C), jnp.float32),
        pltpu.SemaphoreType.DMA,
        pltpu.SemaphoreType.DMA,
    ],
)
def _prop(p_hbm, src_hbm, dst3_hbm, out_hbm, rows0, rows1, sidx, didx,
          acc_sh, sem0, sem1):
    c = lax.axis_index("c")
    s = lax.axis_index("s")
    wid = c * NS + s
    z16 = jnp.zeros((16,), jnp.float32)

    @pl.loop(0, K)
    def _zrows(i):
        for j in range(C // 16):
            rows0[i, pl.ds(j * 16, 16)] = z16

    @pl.loop(0, RPT // K)
    def _zcopy(j):
        pltpu.sync_copy(rows0, acc_sh.at[pl.ds(s * RPT + j * K, K)])

    pltpu.sync_copy(src_hbm.at[pl.ds(wid * EPW, HCH * K)], sidx)
    pltpu.sync_copy(dst3_hbm.at[wid], didx)
    plsc.subcore_barrier()

    def scatter(j, buf):
        pltpu.sync_copy(buf, acc_sh.at[didx.at[j]], add=True)

    def run_phase(pb):
        def gather(j, buf, sem):
            return pltpu.async_copy(
                p_hbm.at[sidx.at[pl.ds((j - pb) * K, K)]], buf, sem)

        def gwait(j, buf, sem):
            pltpu.make_async_copy(
                p_hbm.at[sidx.at[pl.ds((j - pb) * K, K)]], buf, sem).wait()

        gather(pb, rows0, sem0)

        @pl.loop(0, (HCH - 2) // 2)
        def _pipe(i2):
            i = pb + i2 * 2
            gather(i + 1, rows1, sem1)
            gwait(i, rows0, sem0)
            scatter(i, rows0)
            gather(i + 2, rows0, sem0)
            gwait(i + 1, rows1, sem1)
            scatter(i + 1, rows1)

        gather(pb + HCH - 1, rows1, sem1)
        gwait(pb + HCH - 2, rows0, sem0)
        scatter(pb + HCH - 2, rows0)
        gwait(pb + HCH - 1, rows1, sem1)
        scatter(pb + HCH - 1, rows1)

    run_phase(0)
    pltpu.sync_copy(src_hbm.at[pl.ds(wid * EPW + HCH * K, HCH * K)], sidx)
    run_phase(HCH)

    plsc.subcore_barrier()
    pltpu.sync_copy(acc_sh.at[pl.ds(s * RPT, RPT)],
                    out_hbm.at[c, pl.ds(s * RPT, RPT)])


BR = 1000
GRID = N // BR


def _first_body(degp_ref, x_ref, w_ref, p_ref, d_ref):
    deg = degp_ref[0, :, 0:1] + degp_ref[1, :, 0:1] + 1.0
    dv = lax.rsqrt(deg)
    m = jnp.dot(x_ref[...], w_ref[...], preferred_element_type=jnp.float32)
    p_ref[...] = m * dv
    d_ref[...] = dv


def _mid_body(accp_ref, p_ref, d_ref, b_ref, w_ref, pn_ref):
    sm = accp_ref[0] + accp_ref[1] + p_ref[...]
    h = jnp.maximum(sm * d_ref[...] + b_ref[...], 0.0)
    pn_ref[...] = jnp.dot(h, w_ref[...], preferred_element_type=jnp.float32) * d_ref[...]


def _final_body(accp_ref, p_ref, d_ref, b_ref, bt_ref, wl_ref, bl_ref,
                out_ref, pooled, cnt):
    i = pl.program_id(0)

    @pl.when(i == 0)
    def _():
        pooled[...] = jnp.zeros_like(pooled)
        cnt[...] = jnp.zeros_like(cnt)

    sm = accp_ref[0] + accp_ref[1] + p_ref[...]
    h = jnp.maximum(sm * d_ref[...] + b_ref[...], 0.0)
    bt = bt_ref[0]
    gid = lax.broadcasted_iota(jnp.int32, (G, BR), 0)
    oh = (gid == bt).astype(jnp.float32)
    pooled[...] += jnp.dot(oh, h, preferred_element_type=jnp.float32)
    cnt[...] += jnp.broadcast_to(jnp.sum(oh, axis=1, keepdims=True), (G, C))

    @pl.when(i == GRID - 1)
    def _():
        mean = pooled[...] / jnp.maximum(cnt[...], 1.0)
        out_ref[...] = jnp.dot(mean, wl_ref[...],
                               preferred_element_type=jnp.float32) + bl_ref[...]


def _mk_first():
    return pl.pallas_call(
        _first_body,
        grid=(GRID,),
        in_specs=[
            pl.BlockSpec((NC, BR, C), lambda i: (0, i, 0)),
            pl.BlockSpec((BR, C), lambda i: (i, 0)),
            pl.BlockSpec((C, C), lambda i: (0, 0)),
        ],
        out_specs=[
            pl.BlockSpec((BR, C), lambda i: (i, 0)),
            pl.BlockSpec((BR, 1), lambda i: (i, 0)),
        ],
        out_shape=[
            jax.ShapeDtypeStruct((N, C), jnp.float32),
            jax.ShapeDtypeStruct((N, 1), jnp.float32),
        ],
    )


def _mk_mid():
    return pl.pallas_call(
        _mid_body,
        grid=(GRID,),
        in_specs=[
            pl.BlockSpec((NC, BR, C), lambda i: (0, i, 0)),
            pl.BlockSpec((BR, C), lambda i: (i, 0)),
            pl.BlockSpec((BR, 1), lambda i: (i, 0)),
            pl.BlockSpec((1, C), lambda i: (0, 0)),
            pl.BlockSpec((C, C), lambda i: (0, 0)),
        ],
        out_specs=pl.BlockSpec((BR, C), lambda i: (i, 0)),
        out_shape=jax.ShapeDtypeStruct((N, C), jnp.float32),
    )


def _mk_final():
    return pl.pallas_call(
        _final_body,
        grid=(GRID,),
        in_specs=[
            pl.BlockSpec((NC, BR, C), lambda i: (0, i, 0)),
            pl.BlockSpec((BR, C), lambda i: (i, 0)),
            pl.BlockSpec((BR, 1), lambda i: (i, 0)),
            pl.BlockSpec((1, C), lambda i: (0, 0)),
            pl.BlockSpec((1, 1, BR), lambda i: (i, 0, 0)),
            pl.BlockSpec((C, OUT), lambda i: (0, 0)),
            pl.BlockSpec((1, OUT), lambda i: (0, 0)),
        ],
        out_specs=pl.BlockSpec((G, OUT), lambda i: (0, 0)),
        out_shape=jax.ShapeDtypeStruct((G, OUT), jnp.float32),
        scratch_shapes=[
            pltpu.VMEM((G, C), jnp.float32),
            pltpu.VMEM((G, C), jnp.float32),
        ],
    )


def kernel(x, edge_index, batch, W1, b1, W2, b2, W3, b3, Wl, bl):
    npad_e = EPAD - E
    src = jnp.concatenate([edge_index[0],
                           jnp.zeros((npad_e,), jnp.int32)])
    dst3 = jnp.concatenate([edge_index[1],
                            jnp.full((npad_e,), N, jnp.int32)]
                           ).reshape(NW, NCHUNK, K)
    bt3 = batch.reshape(GRID, 1, BR)

    degp = _deg(dst3)

    p1, d = _mk_first()(degp, x, W1)
    acc1 = _prop(p1, src, dst3)
    p2 = _mk_mid()(acc1, p1, d, b1.reshape(1, C), W2)
    acc2 = _prop(p2, src, dst3)
    p3 = _mk_mid()(acc2, p2, d, b2.reshape(1, C), W3)
    acc3 = _prop(p3, src, dst3)
    out = _mk_final()(acc3, p3, d, b3.reshape(1, C), bt3, Wl,
                      bl.reshape(1, OUT))
    return out


# speedup vs baseline: 3.2996x; 3.2996x over previous
import functools

import jax
import jax.numpy as jnp
from jax import lax
from jax.experimental import pallas as pl
from jax.experimental.pallas import tpu as pltpu
from jax.experimental.pallas import tpu_sc as plsc

N = 10000
E = 320000
C = 128
G = 64
OUT = 64

NC = 2
NS = 16
NW = NC * NS
EPW = E // NW
K = 80
NCHUNK = EPW // K
NPAD = 10240
RPT = NPAD // NS

_mesh = plsc.VectorSubcoreMesh(core_axis_name="c", subcore_axis_name="s")


@functools.partial(
    pl.kernel,
    out_type=jax.ShapeDtypeStruct((NC, NPAD, C), jnp.float32),
    mesh=_mesh,
    scratch_types=[
        pltpu.VMEM((K, C), jnp.float32),
        pltpu.VMEM((NCHUNK, K), jnp.int32),
        pltpu.VMEM_SHARED((NPAD, C), jnp.float32),
        pltpu.SemaphoreType.DMA,
        pltpu.SemaphoreType.DMA,
    ],
)
def _deg(dst3_hbm, out_hbm, ones_v, didx, deg_sh, semA, semB):
    c = lax.axis_index("c")
    s = lax.axis_index("s")
    wid = c * NS + s
    z16 = jnp.zeros((16,), jnp.float32)
    ones16 = jnp.full((16,), 1.0, jnp.float32)

    @pl.loop(0, K)
    def _zero(i):
        for j in range(C // 16):
            ones_v[i, pl.ds(j * 16, 16)] = z16

    @pl.loop(0, RPT // K)
    def _zcopy(j):
        pltpu.sync_copy(ones_v, deg_sh.at[pl.ds(s * RPT + j * K, K)])

    pltpu.sync_copy(dst3_hbm.at[wid], didx)
    plsc.subcore_barrier()

    @pl.loop(0, K)
    def _ones(i):
        for j in range(C // 16):
            ones_v[i, pl.ds(j * 16, 16)] = ones16

    GD = 5
    NG = NCHUNK // GD

    def fire(g, sem):
        for t in range(GD):
            pltpu.async_copy(ones_v, deg_sh.at[didx.at[g * GD + t]], sem,
                             add=True)

    def drain(g, sem):
        for t in range(GD):
            pltpu.make_async_copy(ones_v, deg_sh.at[didx.at[g * GD + t]],
                                  sem).wait()

    fire(0, semA)

    @pl.loop(0, (NG - 1) // 2)
    def _grp(t2):
        g = t2 * 2
        fire(g + 1, semB)
        drain(g, semA)
        fire(g + 2, semA)
        drain(g + 1, semB)

    drain(NG - 1, semA)

    plsc.subcore_barrier()
    pltpu.sync_copy(deg_sh.at[pl.ds(s * RPT, RPT)],
                    out_hbm.at[c, pl.ds(s * RPT, RPT)])


@functools.partial(
    pl.kernel,
    out_type=jax.ShapeDtypeStruct((NC, NPAD, C), jnp.float32),
    mesh=_mesh,
    scratch_types=[
        pltpu.VMEM((K, C), jnp.float32),
        pltpu.VMEM((K, C), jnp.float32),
        pltpu.VMEM((EPW,), jnp.int32),
        pltpu.VMEM((NCHUNK, K), jnp.int32),
        pltpu.VMEM_SHARED((NPAD, C), jnp.float32),
        pltpu.SemaphoreType.DMA,
        pltpu.SemaphoreType.DMA,
    ],
)
def _prop(p_hbm, src_hbm, dst3_hbm, out_hbm, rows0, rows1, sidx, didx,
          acc_sh, sem0, sem1):
    c = lax.axis_index("c")
    s = lax.axis_index("s")
    wid = c * NS + s
    z16 = jnp.zeros((16,), jnp.float32)

    @pl.loop(0, K)
    def _zrows(i):
        for j in range(C // 16):
            rows0[i, pl.ds(j * 16, 16)] = z16

    @pl.loop(0, RPT // K)
    def _zcopy(j):
        pltpu.sync_copy(rows0, acc_sh.at[pl.ds(s * RPT + j * K, K)])

    pltpu.sync_copy(src_hbm.at[pl.ds(wid * EPW, EPW)], sidx)
    pltpu.sync_copy(dst3_hbm.at[wid], didx)
    plsc.subcore_barrier()

    def gather(j, buf, sem):
        return pltpu.async_copy(p_hbm.at[sidx.at[pl.ds(j * K, K)]], buf, sem)

    def gwait(j, buf, sem):
        pltpu.make_async_copy(p_hbm.at[sidx.at[pl.ds(j * K, K)]], buf,
                              sem).wait()

    def scatter(j, buf):
        pltpu.sync_copy(buf, acc_sh.at[didx.at[j]], add=True)

    gather(0, rows0, sem0)

    @pl.loop(0, (NCHUNK - 1) // 2)
    def _pipe(i2):
        i = i2 * 2
        gather(i + 1, rows1, sem1)
        gwait(i, rows0, sem0)
        scatter(i, rows0)
        gather(i + 2, rows0, sem0)
        gwait(i + 1, rows1, sem1)
        scatter(i + 1, rows1)

    gwait(NCHUNK - 1, rows0, sem0)
    scatter(NCHUNK - 1, rows0)

    plsc.subcore_barrier()
    pltpu.sync_copy(acc_sh.at[pl.ds(s * RPT, RPT)],
                    out_hbm.at[c, pl.ds(s * RPT, RPT)])


BR = 1000
GRID = N // BR


def _mm_body(x_ref, w_ref, m_ref):
    m_ref[...] = jnp.dot(x_ref[...], w_ref[...],
                         preferred_element_type=jnp.float32)


def _first_body(degp_ref, m_ref, p_ref, d_ref):
    deg = degp_ref[0, :, 0:1] + degp_ref[1, :, 0:1] + 1.0
    dv = lax.rsqrt(deg)
    p_ref[...] = m_ref[...] * dv
    d_ref[...] = dv


def _mid_body(accp_ref, p_ref, d_ref, b_ref, w_ref, pn_ref):
    sm = accp_ref[0] + accp_ref[1] + p_ref[...]
    h = jnp.maximum(sm * d_ref[...] + b_ref[...], 0.0)
    pn_ref[...] = jnp.dot(h, w_ref[...], preferred_element_type=jnp.float32) * d_ref[...]


def _final_body(accp_ref, p_ref, d_ref, b_ref, bt_ref, wl_ref, bl_ref,
                out_ref, pooled, cnt):
    i = pl.program_id(0)

    @pl.when(i == 0)
    def _():
        pooled[...] = jnp.zeros_like(pooled)
        cnt[...] = jnp.zeros_like(cnt)

    sm = accp_ref[0] + accp_ref[1] + p_ref[...]
    h = jnp.maximum(sm * d_ref[...] + b_ref[...], 0.0)
    bt = bt_ref[0]
    gid = lax.broadcasted_iota(jnp.int32, (G, BR), 0)
    oh = (gid == bt).astype(jnp.float32)
    pooled[...] += jnp.dot(oh, h, preferred_element_type=jnp.float32)
    cnt[...] += jnp.broadcast_to(jnp.sum(oh, axis=1, keepdims=True), (G, C))

    @pl.when(i == GRID - 1)
    def _():
        mean = pooled[...] / jnp.maximum(cnt[...], 1.0)
        out_ref[...] = jnp.dot(mean, wl_ref[...],
                               preferred_element_type=jnp.float32) + bl_ref[...]


def _mk_mm():
    return pl.pallas_call(
        _mm_body,
        grid=(GRID,),
        in_specs=[
            pl.BlockSpec((BR, C), lambda i: (i, 0)),
            pl.BlockSpec((C, C), lambda i: (0, 0)),
        ],
        out_specs=pl.BlockSpec((BR, C), lambda i: (i, 0)),
        out_shape=jax.ShapeDtypeStruct((N, C), jnp.float32),
    )


def _mk_first():
    return pl.pallas_call(
        _first_body,
        grid=(GRID,),
        in_specs=[
            pl.BlockSpec((NC, BR, C), lambda i: (0, i, 0)),
            pl.BlockSpec((BR, C), lambda i: (i, 0)),
        ],
        out_specs=[
            pl.BlockSpec((BR, C), lambda i: (i, 0)),
            pl.BlockSpec((BR, 1), lambda i: (i, 0)),
        ],
        out_shape=[
            jax.ShapeDtypeStruct((N, C), jnp.float32),
            jax.ShapeDtypeStruct((N, 1), jnp.float32),
        ],
    )


def _mk_mid():
    return pl.pallas_call(
        _mid_body,
        grid=(GRID,),
        in_specs=[
            pl.BlockSpec((NC, BR, C), lambda i: (0, i, 0)),
            pl.BlockSpec((BR, C), lambda i: (i, 0)),
            pl.BlockSpec((BR, 1), lambda i: (i, 0)),
            pl.BlockSpec((1, C), lambda i: (0, 0)),
            pl.BlockSpec((C, C), lambda i: (0, 0)),
        ],
        out_specs=pl.BlockSpec((BR, C), lambda i: (i, 0)),
        out_shape=jax.ShapeDtypeStruct((N, C), jnp.float32),
    )


def _mk_final():
    return pl.pallas_call(
        _final_body,
        grid=(GRID,),
        in_specs=[
            pl.BlockSpec((NC, BR, C), lambda i: (0, i, 0)),
            pl.BlockSpec((BR, C), lambda i: (i, 0)),
            pl.BlockSpec((BR, 1), lambda i: (i, 0)),
            pl.BlockSpec((1, C), lambda i: (0, 0)),
            pl.BlockSpec((1, 1, BR), lambda i: (i, 0, 0)),
            pl.BlockSpec((C, OUT), lambda i: (0, 0)),
            pl.BlockSpec((1, OUT), lambda i: (0, 0)),
        ],
        out_specs=pl.BlockSpec((G, OUT), lambda i: (0, 0)),
        out_shape=jax.ShapeDtypeStruct((G, OUT), jnp.float32),
        scratch_shapes=[
            pltpu.VMEM((G, C), jnp.float32),
            pltpu.VMEM((G, C), jnp.float32),
        ],
    )


def kernel(x, edge_index, batch, W1, b1, W2, b2, W3, b3, Wl, bl):
    src = edge_index[0]
    dst3 = edge_index[1].reshape(NW, NCHUNK, K)
    bt3 = batch.reshape(GRID, 1, BR)

    degp = _deg(dst3)
    m1 = _mk_mm()(x, W1)

    p1, d = _mk_first()(degp, m1)
    acc1 = _prop(p1, src, dst3)
    p2 = _mk_mid()(acc1, p1, d, b1.reshape(1, C), W2)
    acc2 = _prop(p2, src, dst3)
    p3 = _mk_mid()(acc2, p2, d, b2.reshape(1, C), W3)
    acc3 = _prop(p3, src, dst3)
    out = _mk_final()(acc3, p3, d, b3.reshape(1, C), bt3, Wl,
                      bl.reshape(1, OUT))
    return out


# speedup vs baseline: 3.3570x; 1.0174x over previous
import functools

import jax
import jax.numpy as jnp
from jax import lax
from jax.experimental import pallas as pl
from jax.experimental.pallas import tpu as pltpu
from jax.experimental.pallas import tpu_sc as plsc

N = 10000
E = 320000
C = 128
G = 64
OUT = 64

NC = 2
NS = 16
NW = NC * NS
EPW = E // NW
K = 80
NCHUNK = EPW // K
NPAD = 10240
RPT = NPAD // NS

_mesh = plsc.VectorSubcoreMesh(core_axis_name="c", subcore_axis_name="s")


@functools.partial(
    pl.kernel,
    out_type=jax.ShapeDtypeStruct((NC, NPAD, C), jnp.float32),
    mesh=_mesh,
    scratch_types=[
        pltpu.VMEM((K, C), jnp.float32),
        pltpu.VMEM((NCHUNK, K), jnp.int32),
        pltpu.VMEM_SHARED((NPAD, C), jnp.float32),
        pltpu.SemaphoreType.DMA,
        pltpu.SemaphoreType.DMA,
    ],
)
def _deg(dst3_hbm, out_hbm, ones_v, didx, deg_sh, semA, semB):
    c = lax.axis_index("c")
    s = lax.axis_index("s")
    wid = c * NS + s
    z16 = jnp.zeros((16,), jnp.float32)
    ones16 = jnp.full((16,), 1.0, jnp.float32)

    @pl.loop(0, K)
    def _zero(i):
        for j in range(C // 16):
            ones_v[i, pl.ds(j * 16, 16)] = z16

    @pl.loop(0, RPT // K)
    def _zcopy(j):
        pltpu.sync_copy(ones_v, deg_sh.at[pl.ds(s * RPT + j * K, K)])

    pltpu.sync_copy(dst3_hbm.at[wid], didx)
    plsc.subcore_barrier()

    @pl.loop(0, K)
    def _ones(i):
        for j in range(C // 16):
            ones_v[i, pl.ds(j * 16, 16)] = ones16

    GD = 5
    NG = NCHUNK // GD

    def fire(g, sem):
        for t in range(GD):
            pltpu.async_copy(ones_v, deg_sh.at[didx.at[g * GD + t]], sem,
                             add=True)

    def drain(g, sem):
        for t in range(GD):
            pltpu.make_async_copy(ones_v, deg_sh.at[didx.at[g * GD + t]],
                                  sem).wait()

    fire(0, semA)

    @pl.loop(0, (NG - 1) // 2)
    def _grp(t2):
        g = t2 * 2
        fire(g + 1, semB)
        drain(g, semA)
        fire(g + 2, semA)
        drain(g + 1, semB)

    drain(NG - 1, semA)

    plsc.subcore_barrier()
    pltpu.sync_copy(deg_sh.at[pl.ds(s * RPT, RPT)],
                    out_hbm.at[c, pl.ds(s * RPT, RPT)])


@functools.partial(
    pl.kernel,
    out_type=jax.ShapeDtypeStruct((NC, NPAD, C), jnp.float32),
    mesh=_mesh,
    scratch_types=[
        pltpu.VMEM((K, C), jnp.float32),
        pltpu.VMEM((K, C), jnp.float32),
        pltpu.VMEM((EPW,), jnp.int32),
        pltpu.VMEM((NCHUNK, K), jnp.int32),
        pltpu.VMEM_SHARED((NPAD, C), jnp.float32),
        pltpu.SemaphoreType.DMA,
        pltpu.SemaphoreType.DMA,
    ],
)
def _prop(p_hbm, src_hbm, dst3_hbm, out_hbm, rows0, rows1, sidx, didx,
          acc_sh, sem0, sem1):
    c = lax.axis_index("c")
    s = lax.axis_index("s")
    wid = c * NS + s
    z16 = jnp.zeros((16,), jnp.float32)

    @pl.loop(0, K)
    def _zrows(i):
        for j in range(C // 16):
            rows0[i, pl.ds(j * 16, 16)] = z16

    for j in range(RPT // K):
        pltpu.async_copy(rows0, acc_sh.at[pl.ds(s * RPT + j * K, K)], sem1)
    pltpu.sync_copy(src_hbm.at[pl.ds(wid * EPW, EPW)], sidx)
    pltpu.sync_copy(dst3_hbm.at[wid], didx)
    for j in range(RPT // K):
        pltpu.make_async_copy(rows0, acc_sh.at[pl.ds(s * RPT + j * K, K)],
                              sem1).wait()
    plsc.subcore_barrier()

    def gather(j, buf, sem):
        return pltpu.async_copy(p_hbm.at[sidx.at[pl.ds(j * K, K)]], buf, sem)

    def gwait(j, buf, sem):
        pltpu.make_async_copy(p_hbm.at[sidx.at[pl.ds(j * K, K)]], buf,
                              sem).wait()

    def scatter(j, buf):
        pltpu.sync_copy(buf, acc_sh.at[didx.at[j]], add=True)

    gather(0, rows0, sem0)

    @pl.loop(0, (NCHUNK - 1) // 2)
    def _pipe(i2):
        i = i2 * 2
        gather(i + 1, rows1, sem1)
        gwait(i, rows0, sem0)
        scatter(i, rows0)
        gather(i + 2, rows0, sem0)
        gwait(i + 1, rows1, sem1)
        scatter(i + 1, rows1)

    gwait(NCHUNK - 1, rows0, sem0)
    scatter(NCHUNK - 1, rows0)

    plsc.subcore_barrier()
    pltpu.sync_copy(acc_sh.at[pl.ds(s * RPT, RPT)],
                    out_hbm.at[c, pl.ds(s * RPT, RPT)])


BR = 1000
GRID = N // BR


def _mm_body(x_ref, w_ref, m_ref):
    m_ref[...] = jnp.dot(x_ref[...], w_ref[...],
                         preferred_element_type=jnp.float32)


def _first_body(degp_ref, m_ref, p_ref, d_ref):
    deg = degp_ref[0, :, 0:1] + degp_ref[1, :, 0:1] + 1.0
    dv = lax.rsqrt(deg)
    p_ref[...] = m_ref[...] * dv
    d_ref[...] = dv


def _mid_body(accp_ref, p_ref, d_ref, b_ref, w_ref, pn_ref):
    sm = accp_ref[0] + accp_ref[1] + p_ref[...]
    h = jnp.maximum(sm * d_ref[...] + b_ref[...], 0.0)
    pn_ref[...] = jnp.dot(h, w_ref[...], preferred_element_type=jnp.float32) * d_ref[...]


def _final_body(accp_ref, p_ref, d_ref, b_ref, bt_ref, wl_ref, bl_ref,
                out_ref, pooled, cnt):
    i = pl.program_id(0)

    @pl.when(i == 0)
    def _():
        pooled[...] = jnp.zeros_like(pooled)
        cnt[...] = jnp.zeros_like(cnt)

    sm = accp_ref[0] + accp_ref[1] + p_ref[...]
    h = jnp.maximum(sm * d_ref[...] + b_ref[...], 0.0)
    bt = bt_ref[0]
    gid = lax.broadcasted_iota(jnp.int32, (G, BR), 0)
    oh = (gid == bt).astype(jnp.float32)
    pooled[...] += jnp.dot(oh, h, preferred_element_type=jnp.float32)
    cnt[...] += jnp.broadcast_to(jnp.sum(oh, axis=1, keepdims=True), (G, C))

    @pl.when(i == GRID - 1)
    def _():
        mean = pooled[...] / jnp.maximum(cnt[...], 1.0)
        out_ref[...] = jnp.dot(mean, wl_ref[...],
                               preferred_element_type=jnp.float32) + bl_ref[...]


def _mk_mm():
    return pl.pallas_call(
        _mm_body,
        grid=(GRID,),
        in_specs=[
            pl.BlockSpec((BR, C), lambda i: (i, 0)),
            pl.BlockSpec((C, C), lambda i: (0, 0)),
        ],
        out_specs=pl.BlockSpec((BR, C), lambda i: (i, 0)),
        out_shape=jax.ShapeDtypeStruct((N, C), jnp.float32),
    )


def _mk_first():
    return pl.pallas_call(
        _first_body,
        grid=(GRID,),
        in_specs=[
            pl.BlockSpec((NC, BR, C), lambda i: (0, i, 0)),
            pl.BlockSpec((BR, C), lambda i: (i, 0)),
        ],
        out_specs=[
            pl.BlockSpec((BR, C), lambda i: (i, 0)),
            pl.BlockSpec((BR, 1), lambda i: (i, 0)),
        ],
        out_shape=[
            jax.ShapeDtypeStruct((N, C), jnp.float32),
            jax.ShapeDtypeStruct((N, 1), jnp.float32),
        ],
    )


def _mk_mid():
    return pl.pallas_call(
        _mid_body,
        grid=(GRID,),
        in_specs=[
            pl.BlockSpec((NC, BR, C), lambda i: (0, i, 0)),
            pl.BlockSpec((BR, C), lambda i: (i, 0)),
            pl.BlockSpec((BR, 1), lambda i: (i, 0)),
            pl.BlockSpec((1, C), lambda i: (0, 0)),
            pl.BlockSpec((C, C), lambda i: (0, 0)),
        ],
        out_specs=pl.BlockSpec((BR, C), lambda i: (i, 0)),
        out_shape=jax.ShapeDtypeStruct((N, C), jnp.float32),
    )


def _mk_final():
    return pl.pallas_call(
        _final_body,
        grid=(GRID,),
        in_specs=[
            pl.BlockSpec((NC, BR, C), lambda i: (0, i, 0)),
            pl.BlockSpec((BR, C), lambda i: (i, 0)),
            pl.BlockSpec((BR, 1), lambda i: (i, 0)),
            pl.BlockSpec((1, C), lambda i: (0, 0)),
            pl.BlockSpec((1, 1, BR), lambda i: (i, 0, 0)),
            pl.BlockSpec((C, OUT), lambda i: (0, 0)),
            pl.BlockSpec((1, OUT), lambda i: (0, 0)),
        ],
        out_specs=pl.BlockSpec((G, OUT), lambda i: (0, 0)),
        out_shape=jax.ShapeDtypeStruct((G, OUT), jnp.float32),
        scratch_shapes=[
            pltpu.VMEM((G, C), jnp.float32),
            pltpu.VMEM((G, C), jnp.float32),
        ],
    )


def kernel(x, edge_index, batch, W1, b1, W2, b2, W3, b3, Wl, bl):
    src = edge_index[0]
    dst3 = edge_index[1].reshape(NW, NCHUNK, K)
    bt3 = batch.reshape(GRID, 1, BR)

    degp = _deg(dst3)
    m1 = _mk_mm()(x, W1)

    p1, d = _mk_first()(degp, m1)
    acc1 = _prop(p1, src, dst3)
    p2 = _mk_mid()(acc1, p1, d, b1.reshape(1, C), W2)
    acc2 = _prop(p2, src, dst3)
    p3 = _mk_mid()(acc2, p2, d, b2.reshape(1, C), W3)
    acc3 = _prop(p3, src, dst3)
    out = _mk_final()(acc3, p3, d, b3.reshape(1, C), bt3, Wl,
                      bl.reshape(1, OUT))
    return out
